# baseline (speedup 1.0000x reference)
import functools

import jax
import jax.numpy as jnp
from jax import lax
from jax.experimental import pallas as pl
from jax.experimental.pallas import tpu as pltpu
from jax.experimental.pallas import tpu_sc as plsc

B = 4096
NUM_NUM = 13
N_CAT = 26
VOCAB = 100000
D = 64
NTOK = NUM_NUM + N_CAT

NCORES = 2
NSUB = 16
LANES = 16
NW = NCORES * NSUB
BPW = B // NW
CB = 4
NCHUNK = BPW // CB
IPC = CB * N_CAT
IDX_LEN = BPW * N_CAT
XN_CHUNK = CB * NUM_NUM
XN_PAD = 64
XN_LEN = NCHUNK * XN_PAD
AB_LEN = NUM_NUM * D

_mesh = plsc.VectorSubcoreMesh(core_axis_name="c", subcore_axis_name="s")


@functools.partial(
    pl.kernel,
    out_type=jax.ShapeDtypeStruct((B, NTOK, D), jnp.float32),
    mesh=_mesh,
    compiler_params=pltpu.CompilerParams(use_tc_tiling_on_sc=False),
    scratch_types=[
        pltpu.VMEM((IDX_LEN,), jnp.int32),
        pltpu.VMEM((XN_LEN,), jnp.float32),
        pltpu.VMEM((AB_LEN,), jnp.float32),
        pltpu.VMEM((AB_LEN,), jnp.float32),
        pltpu.VMEM((IPC, D), jnp.float32),
        pltpu.VMEM((CB * NUM_NUM, D), jnp.float32),
        pltpu.SemaphoreType.DMA,
        pltpu.SemaphoreType.DMA,
    ],
)
def _tokenizer(xnum_hbm, xcat_hbm, a_hbm, b_hbm, tab_hbm, out_hbm,
               idx_v, xn_v, a_v, b_v, gbuf, nbuf, gsem, osem):
    wid = lax.axis_index("s") * NCORES + lax.axis_index("c")
    b0 = wid * BPW

    pltpu.sync_copy(xcat_hbm.at[wid], idx_v)
    pltpu.sync_copy(xnum_hbm.at[wid], xn_v)
    pltpu.sync_copy(a_hbm, a_v)
    pltpu.sync_copy(b_hbm, b_v)

    def _off_body(j, _):
        base = j * LANES
        pos = base + lax.iota(jnp.int32, LANES)
        f = lax.rem(pos, N_CAT)
        idx_v[pl.ds(base, LANES)] = idx_v[pl.ds(base, LANES)] + f * VOCAB
        return 0

    lax.fori_loop(0, IDX_LEN // LANES, _off_body, 0)

    def _chunk_body(c, _):
        gh = pltpu.async_copy(
            tab_hbm.at[idx_v.at[pl.ds(c * IPC, IPC)]], gbuf, gsem)

        xvecs = [xn_v[pl.ds(c * XN_PAD + g * LANES, LANES)]
                 for g in range(XN_PAD // LANES)]
        for lb in range(CB):
            for k in range(NUM_NUM):
                s = lb * NUM_NUM + k
                x = xvecs[s // LANES][s % LANES]
                for dslice in range(D // LANES):
                    o = k * D + dslice * LANES
                    nbuf[lb * NUM_NUM + k, pl.ds(dslice * LANES, LANES)] = (
                        a_v[pl.ds(o, LANES)] + b_v[pl.ds(o, LANES)] * x)

        handles = []
        for lb in range(CB):
            row = b0 + c * CB + lb
            handles.append(pltpu.async_copy(
                nbuf.at[pl.ds(lb * NUM_NUM, NUM_NUM)],
                out_hbm.at[row, pl.ds(0, NUM_NUM)], osem))
        gh.wait()
        for lb in range(CB):
            row = b0 + c * CB + lb
            handles.append(pltpu.async_copy(
                gbuf.at[pl.ds(lb * N_CAT, N_CAT)],
                out_hbm.at[row, pl.ds(NUM_NUM, N_CAT)], osem))
        for h in handles:
            h.wait()
        return 0

    lax.fori_loop(0, NCHUNK, _chunk_body, 0)


def kernel(x_num, x_cat, a, b, tables):
    xn = jnp.pad(x_num.reshape(NW, NCHUNK, XN_CHUNK),
                 ((0, 0), (0, 0), (0, XN_PAD - XN_CHUNK)))
    xn = xn.reshape(NW, XN_LEN)
    xc = x_cat.reshape(NW, IDX_LEN)
    af = a.reshape(AB_LEN)
    bf = b.reshape(AB_LEN)
    tab = tables.reshape(N_CAT * VOCAB, D)
    return _tokenizer(xn, xc, af, bf, tab)
